# baseline (speedup 1.0000x reference)
import jax
import jax.numpy as jnp
from jax.experimental import pallas as pl

_B, _T = 16, 4096
_NCH = 4
_CW = _T // _NCH


def _loss_kernel(a_ref, b_ref, lcls_ref, bnd_ref, lbnd_ref, lenc_ref,
                 lenb_ref, total_ref, spoof_ref, bdry_ref):
    col = jax.lax.broadcasted_iota(jnp.int32, (_B, _T), 1)

    a = a_ref[...]
    b = b_ref[...]
    m = jnp.maximum(a, b)
    lse = m + jnp.log(jnp.exp(a - m) + jnp.exp(b - m))
    sel = jnp.where(lcls_ref[...] == 0, a, b)
    ce = lse - sel
    ccond = col < lenc_ref[...]
    csum = jnp.sum(lenc_ref[...]).astype(jnp.float32)
    spoof = jnp.sum(jnp.where(ccond, ce, 0.0)) / (csum + 1e-8)

    pred = bnd_ref[...]
    tgt = lbnd_ref[...]
    bcond = col < lenb_ref[...]
    selp = jnp.where(tgt == 1, pred, 1.0 - pred)
    rawloss = jnp.minimum(-jnp.log(selp), 100.0)
    posb = bcond & (tgt == 1)
    loss = jnp.where(bcond, rawloss, 0.0)
    pos_count = jnp.sum(jnp.where(posb, 1.0, 0.0))
    neg_count_all = jnp.float32(_B * _T) - pos_count
    k = jnp.minimum(neg_count_all, jnp.floor(pos_count * 5.0))
    pos_loss = jnp.sum(jnp.where(posb, rawloss, 0.0))
    neg_vals = jnp.where(posb, 0.0, loss)

    vbits = jax.lax.bitcast_convert_type(neg_vals, jnp.int32)
    k_i = k.astype(jnp.int32)

    def count_ge(mth):
        return jnp.sum((vbits >= mth).astype(jnp.int32))

    def radix_round(lo, s, njs):
        t = jnp.int32(0)
        for j in range(1, njs + 1):
            c = count_ge(lo + (j << s))
            t = t + (c >= k_i).astype(jnp.int32)
        return lo + t * (1 << s)

    lo = radix_round(jnp.int32(0), 29, 2)
    for s in (27, 25, 23, 21, 19, 17, 15, 13, 11, 9, 7, 5, 3, 1):
        lo = radix_round(lo, s, 3)
    lo = radix_round(lo, 0, 1)

    t = jax.lax.bitcast_convert_type(lo, jnp.float32)
    gt = vbits > lo
    cnt_gt = jnp.sum(jnp.where(gt, 1.0, 0.0))
    sum_gt = jnp.sum(jnp.where(gt, neg_vals, 0.0))
    neg_loss = jnp.where(k_i == 0, 0.0, sum_gt + (k - cnt_gt) * t)

    balanced = (pos_loss + neg_loss) / (pos_count + k + 1e-8)
    mean_loss = jnp.sum(loss) / jnp.float32(_B * _T)
    bdry = jnp.where(pos_count == 0.0, mean_loss, balanced)

    total_ref[...] = jnp.broadcast_to(spoof + 0.5 * bdry, (1, 1))
    spoof_ref[...] = jnp.broadcast_to(spoof, (1, 1))
    bdry_ref[...] = jnp.broadcast_to(bdry, (1, 1))


@jax.jit
def kernel(output, boundary, label_cls, label_boundary, len_cls, len_boundary):
    a = output[:, :, 0]
    b = output[:, :, 1]
    lenc = len_cls.reshape(_B, 1)
    lenb = len_boundary.reshape(_B, 1)
    total, spoof, bdry = pl.pallas_call(
        _loss_kernel,
        out_shape=[jax.ShapeDtypeStruct((1, 1), jnp.float32)] * 3,
    )(a, b, label_cls.astype(jnp.int8), boundary,
      label_boundary.astype(jnp.int8), lenc, lenb)
    return (total.reshape(()), spoof.reshape(()), bdry.reshape(()))
